# Initial kernel scaffold; baseline (speedup 1.0000x reference)
import jax
import jax.numpy as jnp
from jax.experimental import pallas as pl


def kernel(x, W_in, b_in, codebook, W_out, b_out):
    raise NotImplementedError("write your pallas kernel here")


# speedup vs baseline: 1.4930x; 1.4930x over previous
import functools

import jax
import jax.numpy as jnp
from jax.experimental import pallas as pl


_COMMIT = 0.25
_KT = 1024


def _vq_body(x_ref, win_ref, bin_ref, cb_ref, ct_ref, wout_ref, bout_ref,
             out_ref, idx_ref, loss_ref, *, n_b, n_tok, commit_scale):
    b = pl.program_id(0)
    T = x_ref.shape[2]
    CD = win_ref.shape[0]
    K = cb_ref.shape[0]

    xb = x_ref[0]
    z = jnp.dot(win_ref[...], xb, preferred_element_type=jnp.float32)
    z = z + bin_ref[...]
    zt = z.T

    run_min = jnp.full((T, 1), jnp.inf, dtype=jnp.float32)
    run_idx = jnp.zeros((T, 1), dtype=jnp.int32)
    run_zq = jnp.zeros((T, CD), dtype=jnp.float32)

    for j in range(K // _KT):
        lo = j * _KT
        ct_tile = ct_ref[:, lo:lo + _KT]
        scores = jnp.dot(zt, ct_tile, preferred_element_type=jnp.float32)
        c2 = jnp.sum(ct_tile * ct_tile, axis=0, keepdims=True)
        dist = c2 - 2.0 * scores
        tmin = jnp.min(dist, axis=1, keepdims=True)
        iota = jax.lax.broadcasted_iota(jnp.int32, (T, _KT), 1) + lo
        targ = jnp.min(jnp.where(dist == tmin, iota, K),
                       axis=1, keepdims=True)
        onehot = (iota == targ).astype(jnp.float32)
        tzq = jnp.dot(onehot, cb_ref[lo:lo + _KT, :],
                      preferred_element_type=jnp.float32)
        upd = tmin < run_min
        run_min = jnp.where(upd, tmin, run_min)
        run_idx = jnp.where(upd, targ, run_idx)
        run_zq = jnp.where(upd, tzq, run_zq)

    outb = jnp.dot(wout_ref[...], run_zq.T,
                   preferred_element_type=jnp.float32) + bout_ref[...]
    out_ref[0] = outb
    idx_ref[...] = run_idx

    diff = run_zq - zt
    part = jnp.sum(diff * diff, axis=(0, 1), keepdims=True)

    @pl.when(b == 0)
    def _init():
        loss_ref[...] = part

    @pl.when(b > 0)
    def _acc():
        loss_ref[...] += part

    @pl.when(b == n_b - 1)
    def _scale():
        loss_ref[...] = loss_ref[...] * commit_scale


@jax.jit
def kernel(x, W_in, b_in, codebook, W_out, b_out):
    B, D, T = x.shape
    K, CD = codebook.shape
    M = B * T

    Ct = codebook.T
    bin2 = b_in.reshape(CD, 1)
    bout2 = b_out.reshape(D, 1)

    body = functools.partial(_vq_body, n_b=B, n_tok=M,
                             commit_scale=_COMMIT / (M * CD))

    out, idx, loss = pl.pallas_call(
        body,
        grid=(B,),
        in_specs=[
            pl.BlockSpec((1, D, T), lambda b: (b, 0, 0)),
            pl.BlockSpec((CD, D), lambda b: (0, 0)),
            pl.BlockSpec((CD, 1), lambda b: (0, 0)),
            pl.BlockSpec((K, CD), lambda b: (0, 0)),
            pl.BlockSpec((CD, K), lambda b: (0, 0)),
            pl.BlockSpec((D, CD), lambda b: (0, 0)),
            pl.BlockSpec((D, 1), lambda b: (0, 0)),
        ],
        out_specs=[
            pl.BlockSpec((1, D, T), lambda b: (b, 0, 0)),
            pl.BlockSpec((T, 1), lambda b: (b, 0)),
            pl.BlockSpec((1, 1), lambda b: (0, 0)),
        ],
        out_shape=[
            jax.ShapeDtypeStruct((B, D, T), jnp.float32),
            jax.ShapeDtypeStruct((M, 1), jnp.int32),
            jax.ShapeDtypeStruct((1, 1), jnp.float32),
        ],
    )(x, W_in, bin2, codebook, Ct, W_out, bout2)

    q = idx.reshape(1, B, T)
    return out, q, loss.reshape(())


# speedup vs baseline: 1.6037x; 1.0742x over previous
import functools

import jax
import jax.numpy as jnp
from jax.experimental import pallas as pl


_COMMIT = 0.25
_KT = 1024
_BIG = 1e9


def _vq_body(x_ref, win_ref, bin_ref, cb_ref, cta_ref, wout_ref, bout_ref,
             out_ref, idx_ref, loss_ref, *, n_b, commit_scale):
    b = pl.program_id(0)
    T = x_ref.shape[2]
    CD = cb_ref.shape[1]
    K = cb_ref.shape[0]

    xb = x_ref[0]
    z = jnp.dot(win_ref[...], xb, preferred_element_type=jnp.float32)
    z = z + bin_ref[...]
    zt_aug = z.T
    zt = zt_aug[:, :CD]

    iota_f = jax.lax.broadcasted_iota(
        jnp.int32, (T, _KT), 1).astype(jnp.float32)

    run_min = jnp.full((T, 1), _BIG, dtype=jnp.float32)
    run_idx = jnp.zeros((T, 1), dtype=jnp.float32)
    run_zq = jnp.zeros((T, CD), dtype=jnp.float32)

    for j in range(K // _KT):
        lo = j * _KT
        scores = jnp.dot(zt, cta_ref[:CD, lo:lo + _KT],
                         preferred_element_type=jnp.float32)
        dist = scores + cta_ref[CD:CD + 1, lo:lo + _KT]
        tmin = jnp.min(dist, axis=1, keepdims=True)
        upd = tmin < run_min
        nm = jnp.where(upd, tmin, run_min)
        key = jnp.where(dist == nm, iota_f, _BIG)
        larg = jnp.min(key, axis=1, keepdims=True)
        onehot = (iota_f == larg).astype(jnp.float32)
        tzq = jnp.dot(onehot, cb_ref[lo:lo + _KT, :],
                      preferred_element_type=jnp.float32)
        run_min = nm
        run_idx = jnp.where(upd, larg + jnp.float32(lo), run_idx)
        run_zq = jnp.where(upd, tzq, run_zq)

    outb = jnp.dot(wout_ref[...], run_zq.T,
                   preferred_element_type=jnp.float32) + bout_ref[...]
    out_ref[0] = outb
    idx_ref[...] = run_idx.astype(jnp.int32)

    diff = run_zq - zt
    part = jnp.sum(diff * diff, axis=(0, 1), keepdims=True)

    @pl.when(b == 0)
    def _init():
        loss_ref[...] = part

    @pl.when(b > 0)
    def _acc():
        loss_ref[...] += part

    @pl.when(b == n_b - 1)
    def _scale():
        loss_ref[...] = loss_ref[...] * commit_scale


@jax.jit
def kernel(x, W_in, b_in, codebook, W_out, b_out):
    B, D, T = x.shape
    K, CD = codebook.shape
    M = B * T

    PAD = 8 - (CD + 1) % 8 if (CD + 1) % 8 else 0
    CA = CD + 1 + PAD
    W_in_aug = jnp.concatenate(
        [W_in, jnp.zeros((1 + PAD, D), jnp.float32)], axis=0)
    b_in_aug = jnp.concatenate(
        [b_in, jnp.ones((1,), jnp.float32), jnp.zeros((PAD,), jnp.float32)])
    c2 = jnp.sum(codebook * codebook, axis=1)
    Ct_aug = jnp.concatenate(
        [-2.0 * codebook.T, c2[None, :], jnp.zeros((PAD, K), jnp.float32)],
        axis=0)

    bin2 = b_in_aug.reshape(CA, 1)
    bout2 = b_out.reshape(D, 1)

    body = functools.partial(_vq_body, n_b=B,
                             commit_scale=_COMMIT / (M * CD))

    out, idx, loss = pl.pallas_call(
        body,
        grid=(B,),
        in_specs=[
            pl.BlockSpec((1, D, T), lambda b: (b, 0, 0)),
            pl.BlockSpec((CA, D), lambda b: (0, 0)),
            pl.BlockSpec((CA, 1), lambda b: (0, 0)),
            pl.BlockSpec((K, CD), lambda b: (0, 0)),
            pl.BlockSpec((CA, K), lambda b: (0, 0)),
            pl.BlockSpec((D, CD), lambda b: (0, 0)),
            pl.BlockSpec((D, 1), lambda b: (0, 0)),
        ],
        out_specs=[
            pl.BlockSpec((1, D, T), lambda b: (b, 0, 0)),
            pl.BlockSpec((T, 1), lambda b: (b, 0)),
            pl.BlockSpec((1, 1), lambda b: (0, 0)),
        ],
        out_shape=[
            jax.ShapeDtypeStruct((B, D, T), jnp.float32),
            jax.ShapeDtypeStruct((M, 1), jnp.int32),
            jax.ShapeDtypeStruct((1, 1), jnp.float32),
        ],
    )(x, W_in_aug, bin2, codebook, Ct_aug, W_out, bout2)

    q = idx.reshape(1, B, T)
    return out, q, loss.reshape(())
